# baseline (speedup 1.0000x reference)
import functools

import jax
import jax.numpy as jnp
from jax import lax
from jax.experimental import pallas as pl
from jax.experimental.pallas import tpu as pltpu
from jax.experimental.pallas import tpu_sc as plsc

N_ROWS = 100000
D = 128
NSEG = 64
G = 128
NC = 2
NS = 16
NW = NC * NS
ROWS_PER_W = N_ROWS // NW
N_BIG = (ROWS_PER_W - 8) // G
NBUF = 6
AHEAD = NBUF - 2
BC_COLS = 12544
BC_GRID = BC_COLS // 128


def _tc_finish(sums_ref, cnts_ref, out_ref):
    s = sums_ref[0] + sums_ref[1]
    c = cnts_ref[0] + cnts_ref[1]
    out_ref[...] = s / jnp.maximum(c, 1.0)


def kernel(x, batch_index):
    bi = batch_index.astype(jnp.int32)
    mesh = plsc.VectorSubcoreMesh(core_axis_name="c", subcore_axis_name="s")

    @functools.partial(
        pl.kernel,
        mesh=mesh,
        out_type=[
            jax.ShapeDtypeStruct((NC, NSEG, D), jnp.float32),
            jax.ShapeDtypeStruct((NC, NSEG, D), jnp.float32),
        ],
        scratch_types=(
            [pltpu.VMEM((G, D), jnp.float32) for _ in range(NBUF)]
            + [pltpu.VMEM((G,), jnp.int32) for _ in range(NBUF)]
            + [
                pltpu.VMEM((G, D), jnp.float32),
                pltpu.VMEM((8, D), jnp.float32),
                pltpu.VMEM((8,), jnp.int32),
                pltpu.VMEM((4, D), jnp.float32),
                pltpu.VMEM_SHARED((NSEG, D), jnp.float32),
                pltpu.VMEM_SHARED((NSEG, D), jnp.float32),
            ]
            + [pltpu.SemaphoreType.DMA for _ in range(4 * NBUF)]
        ),
    )
    def sc_seg(x_hbm, bi_hbm, sums_out, cnts_out, *refs):
        rows_b = refs[0:NBUF]
        idx_b = refs[NBUF:2 * NBUF]
        ones_v, rows8_v, idx8_v, z_v, sums_sh, cnts_sh = (
            refs[2 * NBUF:2 * NBUF + 6])
        sem_gr = refs[2 * NBUF + 6:2 * NBUF + 6 + NBUF]
        sem_gi = refs[2 * NBUF + 6 + NBUF:2 * NBUF + 6 + 2 * NBUF]
        sem_s = refs[2 * NBUF + 6 + 2 * NBUF:2 * NBUF + 6 + 3 * NBUF]
        sem_c = refs[2 * NBUF + 6 + 3 * NBUF:]

        c = lax.axis_index("c")
        s = lax.axis_index("s")
        wid = c * NS + s

        zero16 = jnp.zeros((16,), jnp.float32)
        one16 = jnp.ones((16,), jnp.float32)
        for r in range(4):
            for j in range(D // 16):
                z_v[r, pl.ds(j * 16, 16)] = zero16

        def ones_body(r, carry):
            for j in range(D // 16):
                ones_v[r, pl.ds(j * 16, 16)] = one16
            return carry

        lax.fori_loop(0, G, ones_body, None)

        pltpu.sync_copy(z_v, sums_sh.at[pl.ds(s * 4, 4)])
        pltpu.sync_copy(z_v, cnts_sh.at[pl.ds(s * 4, 4)])
        plsc.subcore_barrier()

        start = (wid * ROWS_PER_W) & -8
        end = jnp.where(wid == NW - 1, N_ROWS, ((wid + 1) * ROWS_PER_W) & -8)
        tail0 = start + N_BIG * G
        n_tail = (end - tail0) // 8

        gathers = {}
        scatters = {}

        def issue_gather(g):
            b = g % NBUF
            off = pl.multiple_of(start + g * G, 8)
            gathers[g] = (
                pltpu.async_copy(x_hbm.at[pl.ds(off, G)], rows_b[b], sem_gr[b]),
                pltpu.async_copy(bi_hbm.at[pl.ds(off, G)], idx_b[b], sem_gi[b]),
            )

        for g in range(AHEAD):
            issue_gather(g)
        for g in range(N_BIG):
            b = g % NBUF
            for d in gathers.pop(g):
                d.wait()
            scatters[g] = (
                pltpu.async_copy(
                    rows_b[b], sums_sh.at[idx_b[b]], sem_s[b], add=True),
                pltpu.async_copy(
                    ones_v, cnts_sh.at[idx_b[b]], sem_c[b], add=True),
            )
            if g + AHEAD < N_BIG:
                prev = g + AHEAD - NBUF
                if prev >= 0:
                    for d in scatters.pop(prev):
                        d.wait()
                issue_gather(g + AHEAD)
        for g in sorted(scatters):
            for d in scatters.pop(g):
                d.wait()

        def tail_body(t, carry):
            off = pl.multiple_of(tail0 + t * 8, 8)
            pltpu.sync_copy(x_hbm.at[pl.ds(off, 8)], rows8_v)
            pltpu.sync_copy(bi_hbm.at[pl.ds(off, 8)], idx8_v)
            pltpu.sync_copy(rows8_v, sums_sh.at[idx8_v], add=True)
            pltpu.sync_copy(
                ones_v.at[pl.ds(0, 8)], cnts_sh.at[idx8_v], add=True)
            return carry

        lax.fori_loop(0, n_tail, tail_body, None)

        plsc.subcore_barrier()

        @pl.when(s == 0)
        def _():
            pltpu.sync_copy(sums_sh, sums_out.at[c])
            pltpu.sync_copy(cnts_sh, cnts_out.at[c])

    sums, cnts = sc_seg(x, bi)

    out = pl.pallas_call(
        _tc_finish,
        out_shape=jax.ShapeDtypeStruct((NSEG, D), jnp.float32),
    )(sums, cnts)
    return out
